# Initial kernel scaffold; baseline (speedup 1.0000x reference)
import jax
import jax.numpy as jnp
from jax.experimental import pallas as pl


def kernel(x, edge_index, W1, a_src1, a_dst1, W2, a_src2, a_dst2):
    raise NotImplementedError("write your pallas kernel here")


# speedup vs baseline: 9.4572x; 9.4572x over previous
import jax
import jax.numpy as jnp
from jax import lax
from jax.experimental import pallas as pl
from jax.experimental.pallas import tpu as pltpu
from jax.experimental.pallas import tpu_sc as plsc

N = 10000
E = 320000
D_IN = 128
HID = 128
NCLS = 16
H = 8

NC = 2
NS = 16
LANES = 16
HALF = N // NC
C = 128
E2 = 327680
EPS = E2 // NS
K = 16
CAP = K + C
SLOC = 40064

f32 = jnp.float32
i32 = jnp.int32

_sc_params = pltpu.CompilerParams(needs_layout_passes=False)
_mesh = plsc.VectorSubcoreMesh(
    core_axis_name="c", subcore_axis_name="s", num_cores=NC, num_subcores=NS
)


def _zero1d(ref, n):
    zf = jnp.zeros((LANES,), ref.dtype)

    def f(i, _):
        ref[pl.ds(i * 16, 16)] = zf
        return 0

    lax.fori_loop(0, n // 16, f, 0)


def _zero2d(ref, rows, cols):
    n16 = cols // 16
    zf = jnp.zeros((LANES,), ref.dtype)

    def f(i, _):
        ref[i // n16, pl.ds((i % n16) * 16, 16)] = zf
        return 0

    lax.fori_loop(0, rows * n16, f, 0)


def _sc_s_body(srcs, dsts, esA, edA, s_hbm, sparts,
               src_v, dst_v, dstc_v, esr, edr, s_loc, sbuf, redacc, redbuf,
               sem1, sem2):
    cid = lax.axis_index("c")
    sid = lax.axis_index("s")
    gid = cid * NS + sid
    half = cid * HALF
    lanes = lax.iota(i32, LANES)
    lt8 = lanes < H
    lm8 = jnp.where(lt8, 1.0, 0.0).astype(f32)

    _zero1d(s_loc, SLOC)
    _zero2d(sbuf, 320, 128)

    def chunk(t, _):
        off = sid * EPS + t * C
        pltpu.sync_copy(srcs.at[pl.ds(off, C)], src_v)
        pltpu.sync_copy(dsts.at[pl.ds(off, C)], dst_v)
        for g in range(C // 16):
            sl = pl.ds(g * 16, 16)
            dstc_v[sl] = jnp.minimum(dst_v[sl], N - 1)
        cp1 = pltpu.async_copy(esA.at[src_v], esr, sem1)
        cp2 = pltpu.async_copy(edA.at[dstc_v], edr, sem2)
        cp1.wait()
        cp2.wait()
        for g in range(C // 16):
            dstg = dst_v[pl.ds(g * 16, 16)]
            hit = (dstg >= half) & (dstg < half + HALF)
            hiti = jnp.where(hit, 1, 0).astype(i32)
            dstl = jnp.where(hit, dstg - half, 0)
            for j2 in range(16):
                j = g * 16 + j2
                e = esr[j, pl.ds(0, 16)] + edr[j, pl.ds(0, 16)]
                e = jnp.where(e > 0, e, 0.2 * e)
                ex = jnp.exp(e) * lm8
                m = lt8 & (lax.broadcast(hiti[j2], (LANES,)) > 0)
                idx = lax.broadcast(dstl[j2] * H, (LANES,)) + lanes
                plsc.addupdate_scatter(s_loc, [idx], ex, mask=m)
        return 0

    lax.fori_loop(0, EPS // C, chunk, 0)
    pltpu.sync_copy(s_loc, sparts.at[gid])
    plsc.subcore_barrier()

    def reduce_stripe(words, outrows):
        basew = sid * 2560
        _zero1d(redacc, 2560)
        for k in range(NS):
            pltpu.sync_copy(
                sparts.at[cid * NS + k].at[pl.ds(basew, words)],
                redbuf.at[pl.ds(0, words)],
            )

            def add(i, _):
                redacc[pl.ds(i * 16, 16)] = (
                    redacc[pl.ds(i * 16, 16)] + redbuf[pl.ds(i * 16, 16)]
                )
                return 0

            lax.fori_loop(0, words // 16, add, 0)

        def pad(i, _):
            v = redacc[pl.ds(i * 16, 16)]
            flat = lax.broadcast(i * 16, (LANES,)) + lanes
            plsc.store_scatter(sbuf, [flat // H, flat % H], v)
            return 0

        lax.fori_loop(0, words // 16, pad, 0)
        pltpu.sync_copy(
            sbuf.at[pl.ds(0, outrows), :],
            s_hbm.at[pl.ds(half + sid * 320, outrows), :],
        )

    @pl.when(sid < NS - 1)
    def _():
        reduce_stripe(2560, 320)

    @pl.when(sid == NS - 1)
    def _():
        reduce_stripe(1664, 200)


def _build_sc_s():
    return pl.kernel(
        _sc_s_body,
        out_type=(
            jax.ShapeDtypeStruct((N, 128), f32),
            jax.ShapeDtypeStruct((NC * NS, SLOC), f32),
        ),
        mesh=_mesh,
        compiler_params=_sc_params,
        scratch_types=[
            pltpu.VMEM((C,), i32),
            pltpu.VMEM((C,), i32),
            pltpu.VMEM((C,), i32),
            pltpu.VMEM((C, 128), f32),
            pltpu.VMEM((C, 128), f32),
            pltpu.VMEM((SLOC,), f32),
            pltpu.VMEM((320, 128), f32),
            pltpu.VMEM((2560,), f32),
            pltpu.VMEM((2560,), f32),
            pltpu.SemaphoreType.DMA,
            pltpu.SemaphoreType.DMA,
        ],
        name="gat_sc_s",
    )


def _build_sc_agg(D):
    ROW = H * D
    if D == 128:
        NB, BUCKET = 10, 512
    else:
        NB, BUCKET = 2, 2560
    LAST = HALF - (NB - 1) * BUCKET
    SR = BUCKET // NS
    QD = max(D // LANES, 1)

    def body(srcs, dsts, esA, edA, z, s_pad, out_hbm, parts,
             src_v, dst_v, pend_s, pend_d, srck, dstk, dstlk,
             esk, edk, srowk, zrows, acc, redbuf, sem1, sem2):
        cid = lax.axis_index("c")
        sid = lax.axis_index("s")
        gid = cid * NS + sid
        half = cid * HALF
        lanes = lax.iota(i32, LANES)
        lm8 = jnp.where(lanes < H, 1.0, 0.0).astype(f32)
        zi16 = jnp.zeros((LANES,), i32)

        for i in range(CAP // 16):
            pend_s[pl.ds(i * 16, 16)] = zi16
            pend_d[pl.ds(i * 16, 16)] = zi16
        dstlk[pl.ds(0, 16)] = zi16
        dstlk[pl.ds(16, 16)] = zi16

        def batch_fire(base_b, valid_cnt):
            srck[pl.ds(0, 16)] = pend_s[pl.ds(0, 16)]
            dstk[pl.ds(0, 16)] = pend_d[pl.ds(0, 16)]
            cpz = pltpu.async_copy(z.at[srck], zrows, sem1)
            cpe = pltpu.async_copy(esA.at[srck], esk, sem2)
            cpd = pltpu.async_copy(edA.at[dstk], edk, sem2)
            cps = pltpu.async_copy(s_pad.at[dstk], srowk, sem2)
            cpz.wait()
            cpe.wait()
            cpd.wait()
            cps.wait()
            vcnt = lax.broadcast(valid_cnt, (LANES,))
            valid = lanes < vcnt
            dstg = dstk[pl.ds(0, 16)]
            dstlk[pl.ds(0, 16)] = jnp.where(valid, dstg - base_b, 0)

            def mul_j(j, _):
                e = esk[j, pl.ds(0, 16)] + edk[j, pl.ds(0, 16)]
                e = jnp.where(e > 0, e, 0.2 * e)
                fv = jnp.where(j < valid_cnt, 1.0, 0.0).astype(f32)
                sden = 8.0 * (srowk[j, pl.ds(0, 16)] + 1e-16)
                w = jnp.exp(e) * lm8 * lax.broadcast(fv, (LANES,)) / sden
                dj = dstlk[pl.ds(j, 16)][0]
                if D == 128:
                    for q in range(QD):
                        vq = jnp.zeros((LANES,), f32)
                        for h in range(H):
                            av = lax.broadcast(w[h], (LANES,))
                            vq = vq + av * zrows[j, pl.ds(h * D + q * 16, 16)]
                        sl = pl.ds(q * 16, 16)
                        acc[dj, sl] = acc[dj, sl] + vq
                else:
                    vq = jnp.zeros((LANES,), f32)
                    for h in range(H):
                        av = lax.broadcast(w[h], (LANES,))
                        vq = vq + av * zrows[j, pl.ds(h * D, 16)]
                    sl = pl.ds(dj * 16, 16)
                    acc[sl] = acc[sl] + vq
                return 0

            lax.fori_loop(0, K, mul_j, 0)

        def bucket(b, _):
            base_b = half + b * BUCKET
            if D == 128:
                _zero2d(acc, BUCKET, D)
            else:
                _zero1d(acc, BUCKET * D)

            def chunk_d(t, cnt):
                off = sid * EPS + t * C
                pltpu.sync_copy(srcs.at[pl.ds(off, C)], src_v)
                pltpu.sync_copy(dsts.at[pl.ds(off, C)], dst_v)
                for g in range(C // 16):
                    srcg = src_v[pl.ds(g * 16, 16)]
                    dstg = dst_v[pl.ds(g * 16, 16)]
                    hit = (
                        (dstg >= base_b)
                        & (dstg < base_b + BUCKET)
                        & (dstg < half + HALF)
                    )
                    keys = jnp.where(hit, lanes, lanes + 16)
                    _, srcc = plsc.sort_key_val(keys, srcg)
                    _, dstc = plsc.sort_key_val(keys, dstg)
                    pend_s[pl.ds(cnt, 16)] = srcc
                    pend_d[pl.ds(cnt, 16)] = dstc
                    pc = plsc.all_reduce_population_count(hit)
                    cnt = cnt + pc[0]

                def fire_body(c):
                    batch_fire(base_b, jnp.int32(K))
                    for i in range(C // 16):
                        pend_s[pl.ds(i * 16, 16)] = pend_s[pl.ds(K + i * 16, 16)]
                        pend_d[pl.ds(i * 16, 16)] = pend_d[pl.ds(K + i * 16, 16)]
                    return c - K

                return lax.while_loop(lambda c: c >= K, fire_body, cnt)

            cnt = lax.fori_loop(0, EPS // C, chunk_d, jnp.int32(0))

            @pl.when(cnt > 0)
            def _():
                batch_fire(base_b, cnt)

            pltpu.sync_copy(acc, parts.at[gid])
            plsc.subcore_barrier()

            base_r = sid * SR
            lastb = jnp.where(b == NB - 1, LAST, BUCKET)
            if D == 128:
                _zero2d(acc, SR, D)
                nd16 = D // 16
                for k in range(NS):
                    pltpu.sync_copy(
                        parts.at[cid * NS + k].at[pl.ds(base_r, SR), :],
                        redbuf,
                    )

                    def add(i, _):
                        r = i // nd16
                        c = (i % nd16) * 16
                        acc[r, pl.ds(c, 16)] = (
                            acc[r, pl.ds(c, 16)] + redbuf[r, pl.ds(c, 16)]
                        )
                        return 0

                    lax.fori_loop(0, SR * D // 16, add, 0)

                def dump(i, _):
                    @pl.when(base_r + i * 8 < lastb)
                    def _():
                        pltpu.sync_copy(
                            acc.at[pl.ds(i * 8, 8), :],
                            out_hbm.at[pl.ds(base_b + base_r + i * 8, 8), :],
                        )
                    return 0

                lax.fori_loop(0, SR // 8, dump, 0)
            else:
                _zero1d(acc, SR * 16)
                for k in range(NS):
                    pltpu.sync_copy(
                        parts.at[cid * NS + k].at[pl.ds(base_r * 16, SR * 16)],
                        redbuf,
                    )

                    def add(i, _):
                        acc[pl.ds(i * 16, 16)] = (
                            acc[pl.ds(i * 16, 16)] + redbuf[pl.ds(i * 16, 16)]
                        )
                        return 0

                    lax.fori_loop(0, SR, add, 0)

                def dump(i, _):
                    @pl.when(base_r + i * 8 < lastb)
                    def _():
                        pltpu.sync_copy(
                            acc.at[pl.ds(i * 128, 128)],
                            out_hbm.at[
                                pl.ds((base_b + base_r + i * 8) * 16, 128)
                            ],
                        )
                    return 0

                lax.fori_loop(0, SR // 8, dump, 0)
            plsc.subcore_barrier()
            return 0

        lax.fori_loop(0, NB, bucket, 0)

    if D == 128:
        out_sds = jax.ShapeDtypeStruct((N, D), f32)
        parts_sds = jax.ShapeDtypeStruct((NC * NS, BUCKET, D), f32)
        acc_t = pltpu.VMEM((BUCKET, D), f32)
        red_t = pltpu.VMEM((SR, D), f32)
    else:
        out_sds = jax.ShapeDtypeStruct((N * 16,), f32)
        parts_sds = jax.ShapeDtypeStruct((NC * NS, BUCKET * 16), f32)
        acc_t = pltpu.VMEM((BUCKET * 16,), f32)
        red_t = pltpu.VMEM((SR * 16,), f32)
    return pl.kernel(
        body,
        out_type=(out_sds, parts_sds),
        mesh=_mesh,
        compiler_params=_sc_params,
        scratch_types=[
            pltpu.VMEM((C,), i32),
            pltpu.VMEM((C,), i32),
            pltpu.VMEM((CAP,), i32),
            pltpu.VMEM((CAP,), i32),
            pltpu.VMEM((K,), i32),
            pltpu.VMEM((K,), i32),
            pltpu.VMEM((2 * K,), i32),
            pltpu.VMEM((K, 128), f32),
            pltpu.VMEM((K, 128), f32),
            pltpu.VMEM((K, 128), f32),
            pltpu.VMEM((K, ROW), f32),
            acc_t,
            red_t,
            pltpu.SemaphoreType.DMA,
            pltpu.SemaphoreType.DMA,
        ],
        name=f"gat_sc_agg_d{D}",
    )


_sc_s = _build_sc_s()
_sc_agg1 = _build_sc_agg(HID)
_sc_agg2 = _build_sc_agg(NCLS)

BN = 400
GRID = N // BN


def _tc1_body(x_ref, w_ref, a_ref, z_ref, es_ref, ed_ref):
    zv = jnp.dot(x_ref[...], w_ref[...], preferred_element_type=f32)
    z_ref[...] = zv
    zh = zv.reshape(BN, H, HID)
    a = a_ref[...]
    zpad = jnp.zeros((BN, 128 - H), f32)
    es = jnp.sum(zh * a[None, :H, :], axis=-1)
    ed = jnp.sum(zh * a[None, H:, :], axis=-1)
    es_ref[...] = jnp.concatenate([es, zpad], axis=1)
    ed_ref[...] = jnp.concatenate([ed, zpad], axis=1)


def _tc2_body(h_ref, w_ref, a_ref, z_ref, es_ref, ed_ref):
    hmean = h_ref[...]
    hact = jnp.where(hmean > 0, hmean, jnp.exp(hmean) - 1.0)
    zv = jnp.dot(hact, w_ref[...], preferred_element_type=f32)
    z_ref[...] = zv
    zh = zv.reshape(BN, H, NCLS)
    a = a_ref[...]
    zpad = jnp.zeros((BN, 128 - H), f32)
    es = jnp.sum(zh * a[None, :H, :], axis=-1)
    ed = jnp.sum(zh * a[None, H:, :], axis=-1)
    es_ref[...] = jnp.concatenate([es, zpad], axis=1)
    ed_ref[...] = jnp.concatenate([ed, zpad], axis=1)


def _tc3_body(o_ref, out_ref):
    hmean = o_ref[...]
    m = jnp.max(hmean, axis=-1, keepdims=True)
    ev = jnp.exp(hmean - m)
    out_ref[...] = ev / jnp.sum(ev, axis=-1, keepdims=True)


def _row_spec(cols):
    return pl.BlockSpec((BN, cols), lambda i: (i, 0))


def _full_spec(r, c):
    return pl.BlockSpec((r, c), lambda i: (0, 0))


_tc1 = pl.pallas_call(
    _tc1_body,
    grid=(GRID,),
    in_specs=[_row_spec(D_IN), _full_spec(D_IN, H * HID), _full_spec(2 * H, HID)],
    out_specs=[_row_spec(H * HID), _row_spec(128), _row_spec(128)],
    out_shape=[
        jax.ShapeDtypeStruct((N, H * HID), f32),
        jax.ShapeDtypeStruct((N, 128), f32),
        jax.ShapeDtypeStruct((N, 128), f32),
    ],
)

_tc2 = pl.pallas_call(
    _tc2_body,
    grid=(GRID,),
    in_specs=[
        _row_spec(HID),
        _full_spec(HID, H * NCLS),
        _full_spec(2 * H, NCLS),
    ],
    out_specs=[_row_spec(H * NCLS), _row_spec(128), _row_spec(128)],
    out_shape=[
        jax.ShapeDtypeStruct((N, H * NCLS), f32),
        jax.ShapeDtypeStruct((N, 128), f32),
        jax.ShapeDtypeStruct((N, 128), f32),
    ],
)

_tc3 = pl.pallas_call(
    _tc3_body,
    grid=(GRID,),
    in_specs=[_row_spec(NCLS)],
    out_specs=_row_spec(NCLS),
    out_shape=jax.ShapeDtypeStruct((N, NCLS), f32),
)


@jax.jit
def kernel(x, edge_index, W1, a_src1, a_dst1, W2, a_src2, a_dst2):
    srcs = jnp.concatenate(
        [edge_index[0].astype(i32), jnp.zeros((E2 - E,), i32)]
    )
    dsts = jnp.concatenate(
        [edge_index[1].astype(i32), jnp.full((E2 - E,), N, i32)]
    )
    a1 = jnp.concatenate([a_src1, a_dst1], axis=0)
    a2 = jnp.concatenate([a_src2, a_dst2], axis=0)
    z1, es1, ed1 = _tc1(x, W1, a1)
    s1, _ = _sc_s(srcs, dsts, es1, ed1)
    h1, _ = _sc_agg1(srcs, dsts, es1, ed1, z1, s1)
    z2, es2, ed2 = _tc2(h1, W2, a2)
    s2, _ = _sc_s(srcs, dsts, es2, ed2)
    o2f, _ = _sc_agg2(srcs, dsts, es2, ed2, z2, s2)
    return _tc3(o2f.reshape(N, NCLS))


# speedup vs baseline: 9.6761x; 1.0231x over previous
import jax
import jax.numpy as jnp
from jax import lax
from jax.experimental import pallas as pl
from jax.experimental.pallas import tpu as pltpu
from jax.experimental.pallas import tpu_sc as plsc

N = 10000
E = 320000
D_IN = 128
HID = 128
NCLS = 16
H = 8

NC = 2
NS = 16
LANES = 16
HALF = N // NC
C = 128
E2 = 327680
EPS = E2 // NS
K = 16
CAP = K + C
SLOC = 40064

f32 = jnp.float32
i32 = jnp.int32

_sc_params = pltpu.CompilerParams(needs_layout_passes=False)
_mesh = plsc.VectorSubcoreMesh(
    core_axis_name="c", subcore_axis_name="s", num_cores=NC, num_subcores=NS
)


def _zero1d(ref, n):
    zf = jnp.zeros((LANES,), ref.dtype)

    def f(i, _):
        ref[pl.ds(i * 16, 16)] = zf
        return 0

    lax.fori_loop(0, n // 16, f, 0)


def _zero2d(ref, rows, cols):
    n16 = cols // 16
    zf = jnp.zeros((LANES,), ref.dtype)

    def f(i, _):
        ref[i // n16, pl.ds((i % n16) * 16, 16)] = zf
        return 0

    lax.fori_loop(0, rows * n16, f, 0)


def _sc_s_body(srcs, dsts, esA, edA, s_hbm, sparts,
               src_v, dst_v, dstc_v, esr, edr, s_loc, sbuf, redacc, redbuf,
               sem1, sem2):
    cid = lax.axis_index("c")
    sid = lax.axis_index("s")
    gid = cid * NS + sid
    half = cid * HALF
    lanes = lax.iota(i32, LANES)
    lt8 = lanes < H
    lm8 = jnp.where(lt8, 1.0, 0.0).astype(f32)

    _zero1d(s_loc, SLOC)
    _zero2d(sbuf, 320, 128)

    def chunk(t, _):
        off = sid * EPS + t * C
        pltpu.sync_copy(srcs.at[pl.ds(off, C)], src_v)
        pltpu.sync_copy(dsts.at[pl.ds(off, C)], dst_v)
        for g in range(C // 16):
            sl = pl.ds(g * 16, 16)
            dstc_v[sl] = jnp.minimum(dst_v[sl], N - 1)
        cp1 = pltpu.async_copy(esA.at[src_v], esr, sem1)
        cp2 = pltpu.async_copy(edA.at[dstc_v], edr, sem2)
        cp1.wait()
        cp2.wait()
        for g in range(C // 16):
            dstg = dst_v[pl.ds(g * 16, 16)]
            hit = (dstg >= half) & (dstg < half + HALF)
            hiti = jnp.where(hit, 1, 0).astype(i32)
            dstl = jnp.where(hit, dstg - half, 0)
            for j2 in range(16):
                j = g * 16 + j2
                e = esr[j, pl.ds(0, 16)] + edr[j, pl.ds(0, 16)]
                e = jnp.where(e > 0, e, 0.2 * e)
                ex = jnp.exp(e) * lm8
                m = lt8 & (lax.broadcast(hiti[j2], (LANES,)) > 0)
                idx = lax.broadcast(dstl[j2] * H, (LANES,)) + lanes
                plsc.addupdate_scatter(s_loc, [idx], ex, mask=m)
        return 0

    lax.fori_loop(0, EPS // C, chunk, 0)
    pltpu.sync_copy(s_loc, sparts.at[gid])
    plsc.subcore_barrier()

    def reduce_stripe(words, outrows):
        basew = sid * 2560
        _zero1d(redacc, 2560)
        for k in range(NS):
            pltpu.sync_copy(
                sparts.at[cid * NS + k].at[pl.ds(basew, words)],
                redbuf.at[pl.ds(0, words)],
            )

            def add(i, _):
                redacc[pl.ds(i * 16, 16)] = (
                    redacc[pl.ds(i * 16, 16)] + redbuf[pl.ds(i * 16, 16)]
                )
                return 0

            lax.fori_loop(0, words // 16, add, 0)

        def pad(i, _):
            v = redacc[pl.ds(i * 16, 16)]
            flat = lax.broadcast(i * 16, (LANES,)) + lanes
            plsc.store_scatter(sbuf, [flat // H, flat % H], v)
            return 0

        lax.fori_loop(0, words // 16, pad, 0)
        pltpu.sync_copy(
            sbuf.at[pl.ds(0, outrows), :],
            s_hbm.at[pl.ds(half + sid * 320, outrows), :],
        )

    @pl.when(sid < NS - 1)
    def _():
        reduce_stripe(2560, 320)

    @pl.when(sid == NS - 1)
    def _():
        reduce_stripe(1664, 200)


def _build_sc_s():
    return pl.kernel(
        _sc_s_body,
        out_type=(
            jax.ShapeDtypeStruct((N, 128), f32),
            jax.ShapeDtypeStruct((NC * NS, SLOC), f32),
        ),
        mesh=_mesh,
        compiler_params=_sc_params,
        scratch_types=[
            pltpu.VMEM((C,), i32),
            pltpu.VMEM((C,), i32),
            pltpu.VMEM((C,), i32),
            pltpu.VMEM((C, 128), f32),
            pltpu.VMEM((C, 128), f32),
            pltpu.VMEM((SLOC,), f32),
            pltpu.VMEM((320, 128), f32),
            pltpu.VMEM((2560,), f32),
            pltpu.VMEM((2560,), f32),
            pltpu.SemaphoreType.DMA,
            pltpu.SemaphoreType.DMA,
        ],
        name="gat_sc_s",
    )


def _build_sc_agg(D):
    ROW = H * D
    if D == 128:
        NB, BUCKET = 8, 640
    else:
        NB, BUCKET = 2, 2560
    LAST = HALF - (NB - 1) * BUCKET
    SR = BUCKET // NS
    QD = max(D // LANES, 1)

    def body(srcs, dsts, esA, edA, z, s_pad, out_hbm, parts,
             src_v, dst_v, pend_s, srck, dstk, dstlk,
             esk, edk, srowk, zrows, acc, redbuf, sem1, sem2):
        cid = lax.axis_index("c")
        sid = lax.axis_index("s")
        gid = cid * NS + sid
        half = cid * HALF
        lanes = lax.iota(i32, LANES)
        lm8 = jnp.where(lanes < H, 1.0, 0.0).astype(f32)
        zi16 = jnp.zeros((LANES,), i32)

        for i in range(CAP // 16):
            pend_s[pl.ds(i * 16, 16)] = zi16
        dstlk[pl.ds(0, 16)] = zi16
        dstlk[pl.ds(16, 16)] = zi16

        def batch_fire(base_b, valid_cnt):
            pk = pend_s[pl.ds(0, 16)]
            srck[pl.ds(0, 16)] = pk // 16384
            dstk[pl.ds(0, 16)] = pk % 16384
            cpz = pltpu.async_copy(z.at[srck], zrows, sem1)
            cpe = pltpu.async_copy(esA.at[srck], esk, sem2)
            cpd = pltpu.async_copy(edA.at[dstk], edk, sem2)
            cps = pltpu.async_copy(s_pad.at[dstk], srowk, sem2)
            cpz.wait()
            cpe.wait()
            cpd.wait()
            cps.wait()
            vcnt = lax.broadcast(valid_cnt, (LANES,))
            valid = lanes < vcnt
            dstg = dstk[pl.ds(0, 16)]
            dstlk[pl.ds(0, 16)] = jnp.where(valid, dstg - base_b, 0)

            def mul_j(j, _):
                e = esk[j, pl.ds(0, 16)] + edk[j, pl.ds(0, 16)]
                e = jnp.where(e > 0, e, 0.2 * e)
                fv = jnp.where(j < valid_cnt, 1.0, 0.0).astype(f32)
                sden = 8.0 * (srowk[j, pl.ds(0, 16)] + 1e-16)
                w = jnp.exp(e) * lm8 * lax.broadcast(fv, (LANES,)) / sden
                dj = dstlk[pl.ds(j, 16)][0]
                if D == 128:
                    for q in range(QD):
                        vq = jnp.zeros((LANES,), f32)
                        for h in range(H):
                            av = lax.broadcast(w[h], (LANES,))
                            vq = vq + av * zrows[j, pl.ds(h * D + q * 16, 16)]
                        sl = pl.ds(q * 16, 16)
                        acc[dj, sl] = acc[dj, sl] + vq
                else:
                    vq = jnp.zeros((LANES,), f32)
                    for h in range(H):
                        av = lax.broadcast(w[h], (LANES,))
                        vq = vq + av * zrows[j, pl.ds(h * D, 16)]
                    sl = pl.ds(dj * 16, 16)
                    acc[sl] = acc[sl] + vq
                return 0

            lax.fori_loop(0, K, mul_j, 0)

        def bucket(b, _):
            base_b = half + b * BUCKET
            if D == 128:
                _zero2d(acc, BUCKET, D)
            else:
                _zero1d(acc, BUCKET * D)

            def chunk_d(t, cnt):
                off = sid * EPS + t * C
                pltpu.sync_copy(srcs.at[pl.ds(off, C)], src_v)
                pltpu.sync_copy(dsts.at[pl.ds(off, C)], dst_v)
                for g in range(C // 16):
                    srcg = src_v[pl.ds(g * 16, 16)]
                    dstg = dst_v[pl.ds(g * 16, 16)]
                    hit = (
                        (dstg >= base_b)
                        & (dstg < base_b + BUCKET)
                        & (dstg < half + HALF)
                    )
                    pc = plsc.all_reduce_population_count(hit)

                    @pl.when(pc[0] > 0)
                    def _():
                        keys = jnp.where(hit, lanes, lanes + 16)
                        packed = srcg * 16384 + dstg
                        _, pkc = plsc.sort_key_val(keys, packed)
                        pend_s[pl.ds(cnt, 16)] = pkc

                    cnt = cnt + pc[0]

                def fire_body(c):
                    batch_fire(base_b, jnp.int32(K))
                    for i in range(C // 16):
                        pend_s[pl.ds(i * 16, 16)] = pend_s[pl.ds(K + i * 16, 16)]
                    return c - K

                return lax.while_loop(lambda c: c >= K, fire_body, cnt)

            cnt = lax.fori_loop(0, EPS // C, chunk_d, jnp.int32(0))

            @pl.when(cnt > 0)
            def _():
                batch_fire(base_b, cnt)

            pltpu.sync_copy(acc, parts.at[gid])
            plsc.subcore_barrier()

            base_r = sid * SR
            lastb = jnp.where(b == NB - 1, LAST, BUCKET)
            if D == 128:
                _zero2d(acc, SR, D)
                nd16 = D // 16
                for k in range(NS):
                    pltpu.sync_copy(
                        parts.at[cid * NS + k].at[pl.ds(base_r, SR), :],
                        redbuf,
                    )

                    def add(i, _):
                        r = i // nd16
                        c = (i % nd16) * 16
                        acc[r, pl.ds(c, 16)] = (
                            acc[r, pl.ds(c, 16)] + redbuf[r, pl.ds(c, 16)]
                        )
                        return 0

                    lax.fori_loop(0, SR * D // 16, add, 0)

                def dump(i, _):
                    @pl.when(base_r + i * 8 < lastb)
                    def _():
                        pltpu.sync_copy(
                            acc.at[pl.ds(i * 8, 8), :],
                            out_hbm.at[pl.ds(base_b + base_r + i * 8, 8), :],
                        )
                    return 0

                lax.fori_loop(0, SR // 8, dump, 0)
            else:
                _zero1d(acc, SR * 16)
                for k in range(NS):
                    pltpu.sync_copy(
                        parts.at[cid * NS + k].at[pl.ds(base_r * 16, SR * 16)],
                        redbuf,
                    )

                    def add(i, _):
                        acc[pl.ds(i * 16, 16)] = (
                            acc[pl.ds(i * 16, 16)] + redbuf[pl.ds(i * 16, 16)]
                        )
                        return 0

                    lax.fori_loop(0, SR, add, 0)

                def dump(i, _):
                    @pl.when(base_r + i * 8 < lastb)
                    def _():
                        pltpu.sync_copy(
                            acc.at[pl.ds(i * 128, 128)],
                            out_hbm.at[
                                pl.ds((base_b + base_r + i * 8) * 16, 128)
                            ],
                        )
                    return 0

                lax.fori_loop(0, SR // 8, dump, 0)
            plsc.subcore_barrier()
            return 0

        lax.fori_loop(0, NB, bucket, 0)

    if D == 128:
        out_sds = jax.ShapeDtypeStruct((N, D), f32)
        parts_sds = jax.ShapeDtypeStruct((NC * NS, BUCKET, D), f32)
        acc_t = pltpu.VMEM((BUCKET, D), f32)
        red_t = pltpu.VMEM((SR, D), f32)
    else:
        out_sds = jax.ShapeDtypeStruct((N * 16,), f32)
        parts_sds = jax.ShapeDtypeStruct((NC * NS, BUCKET * 16), f32)
        acc_t = pltpu.VMEM((BUCKET * 16,), f32)
        red_t = pltpu.VMEM((SR * 16,), f32)
    return pl.kernel(
        body,
        out_type=(out_sds, parts_sds),
        mesh=_mesh,
        compiler_params=_sc_params,
        scratch_types=[
            pltpu.VMEM((C,), i32),
            pltpu.VMEM((C,), i32),
            pltpu.VMEM((CAP,), i32),
            pltpu.VMEM((K,), i32),
            pltpu.VMEM((K,), i32),
            pltpu.VMEM((2 * K,), i32),
            pltpu.VMEM((K, 128), f32),
            pltpu.VMEM((K, 128), f32),
            pltpu.VMEM((K, 128), f32),
            pltpu.VMEM((K, ROW), f32),
            acc_t,
            red_t,
            pltpu.SemaphoreType.DMA,
            pltpu.SemaphoreType.DMA,
        ],
        name=f"gat_sc_agg_d{D}",
    )


_sc_s = _build_sc_s()
_sc_agg1 = _build_sc_agg(HID)
_sc_agg2 = _build_sc_agg(NCLS)

BN = 400
GRID = N // BN


def _tc1_body(x_ref, w_ref, a_ref, z_ref, es_ref, ed_ref):
    zv = jnp.dot(x_ref[...], w_ref[...], preferred_element_type=f32)
    z_ref[...] = zv
    zh = zv.reshape(BN, H, HID)
    a = a_ref[...]
    zpad = jnp.zeros((BN, 128 - H), f32)
    es = jnp.sum(zh * a[None, :H, :], axis=-1)
    ed = jnp.sum(zh * a[None, H:, :], axis=-1)
    es_ref[...] = jnp.concatenate([es, zpad], axis=1)
    ed_ref[...] = jnp.concatenate([ed, zpad], axis=1)


def _tc2_body(h_ref, w_ref, a_ref, z_ref, es_ref, ed_ref):
    hmean = h_ref[...]
    hact = jnp.where(hmean > 0, hmean, jnp.exp(hmean) - 1.0)
    zv = jnp.dot(hact, w_ref[...], preferred_element_type=f32)
    z_ref[...] = zv
    zh = zv.reshape(BN, H, NCLS)
    a = a_ref[...]
    zpad = jnp.zeros((BN, 128 - H), f32)
    es = jnp.sum(zh * a[None, :H, :], axis=-1)
    ed = jnp.sum(zh * a[None, H:, :], axis=-1)
    es_ref[...] = jnp.concatenate([es, zpad], axis=1)
    ed_ref[...] = jnp.concatenate([ed, zpad], axis=1)


def _tc3_body(o_ref, out_ref):
    hmean = o_ref[...]
    m = jnp.max(hmean, axis=-1, keepdims=True)
    ev = jnp.exp(hmean - m)
    out_ref[...] = ev / jnp.sum(ev, axis=-1, keepdims=True)


def _row_spec(cols):
    return pl.BlockSpec((BN, cols), lambda i: (i, 0))


def _full_spec(r, c):
    return pl.BlockSpec((r, c), lambda i: (0, 0))


_tc1 = pl.pallas_call(
    _tc1_body,
    grid=(GRID,),
    in_specs=[_row_spec(D_IN), _full_spec(D_IN, H * HID), _full_spec(2 * H, HID)],
    out_specs=[_row_spec(H * HID), _row_spec(128), _row_spec(128)],
    out_shape=[
        jax.ShapeDtypeStruct((N, H * HID), f32),
        jax.ShapeDtypeStruct((N, 128), f32),
        jax.ShapeDtypeStruct((N, 128), f32),
    ],
)

_tc2 = pl.pallas_call(
    _tc2_body,
    grid=(GRID,),
    in_specs=[
        _row_spec(HID),
        _full_spec(HID, H * NCLS),
        _full_spec(2 * H, NCLS),
    ],
    out_specs=[_row_spec(H * NCLS), _row_spec(128), _row_spec(128)],
    out_shape=[
        jax.ShapeDtypeStruct((N, H * NCLS), f32),
        jax.ShapeDtypeStruct((N, 128), f32),
        jax.ShapeDtypeStruct((N, 128), f32),
    ],
)

_tc3 = pl.pallas_call(
    _tc3_body,
    grid=(GRID,),
    in_specs=[_row_spec(NCLS)],
    out_specs=_row_spec(NCLS),
    out_shape=jax.ShapeDtypeStruct((N, NCLS), f32),
)


@jax.jit
def kernel(x, edge_index, W1, a_src1, a_dst1, W2, a_src2, a_dst2):
    srcs = jnp.concatenate(
        [edge_index[0].astype(i32), jnp.zeros((E2 - E,), i32)]
    )
    dsts = jnp.concatenate(
        [edge_index[1].astype(i32), jnp.full((E2 - E,), N, i32)]
    )
    a1 = jnp.concatenate([a_src1, a_dst1], axis=0)
    a2 = jnp.concatenate([a_src2, a_dst2], axis=0)
    z1, es1, ed1 = _tc1(x, W1, a1)
    s1, _ = _sc_s(srcs, dsts, es1, ed1)
    h1, _ = _sc_agg1(srcs, dsts, es1, ed1, z1, s1)
    z2, es2, ed2 = _tc2(h1, W2, a2)
    s2, _ = _sc_s(srcs, dsts, es2, ed2)
    o2f, _ = _sc_agg2(srcs, dsts, es2, ed2, z2, s2)
    return _tc3(o2f.reshape(N, NCLS))


# speedup vs baseline: 9.7210x; 1.0046x over previous
import jax
import jax.numpy as jnp
from jax import lax
from jax.experimental import pallas as pl
from jax.experimental.pallas import tpu as pltpu
from jax.experimental.pallas import tpu_sc as plsc

N = 10000
E = 320000
D_IN = 128
HID = 128
NCLS = 16
H = 8

NC = 2
NS = 16
LANES = 16
HALF = N // NC
C = 128
E2 = 327680
EPS = E2 // NS
K = 16
CAP = K + C
SLOC = 40064

f32 = jnp.float32
i32 = jnp.int32

_sc_params = pltpu.CompilerParams(needs_layout_passes=False)
_mesh = plsc.VectorSubcoreMesh(
    core_axis_name="c", subcore_axis_name="s", num_cores=NC, num_subcores=NS
)


def _zero1d(ref, n):
    zf = jnp.zeros((LANES,), ref.dtype)

    def f(i, _):
        ref[pl.ds(i * 16, 16)] = zf
        return 0

    lax.fori_loop(0, n // 16, f, 0)


def _zero2d(ref, rows, cols):
    n16 = cols // 16
    zf = jnp.zeros((LANES,), ref.dtype)

    def f(i, _):
        ref[i // n16, pl.ds((i % n16) * 16, 16)] = zf
        return 0

    lax.fori_loop(0, rows * n16, f, 0)


def _sc_s_body(srcs, dsts, esA, edA, s_hbm, sparts,
               src_v, dst_v, dstc_v, esr, edr, s_loc, sbuf, redacc, redbuf,
               sem1, sem2):
    cid = lax.axis_index("c")
    sid = lax.axis_index("s")
    gid = cid * NS + sid
    half = cid * HALF
    lanes = lax.iota(i32, LANES)
    lt8 = lanes < H
    lm8 = jnp.where(lt8, 1.0, 0.0).astype(f32)

    _zero1d(s_loc, SLOC)

    def chunk(t, _):
        off = sid * EPS + t * C
        pltpu.sync_copy(srcs.at[pl.ds(off, C)], src_v)
        pltpu.sync_copy(dsts.at[pl.ds(off, C)], dst_v)
        for g in range(C // 16):
            sl = pl.ds(g * 16, 16)
            dstc_v[sl] = jnp.minimum(dst_v[sl], N - 1)
        cp1 = pltpu.async_copy(esA.at[src_v], esr, sem1)
        cp2 = pltpu.async_copy(edA.at[dstc_v], edr, sem2)
        cp1.wait()
        cp2.wait()
        for g in range(C // 16):
            dstg = dst_v[pl.ds(g * 16, 16)]
            hit = (dstg >= half) & (dstg < half + HALF)
            hiti = jnp.where(hit, 1, 0).astype(i32)
            dstl = jnp.where(hit, dstg - half, 0)
            for j2 in range(16):
                j = g * 16 + j2
                e = esr[j, pl.ds(0, 16)] + edr[j, pl.ds(0, 16)]
                e = jnp.where(e > 0, e, 0.2 * e)
                ex = jnp.exp(e) * lm8
                m = lt8 & (lax.broadcast(hiti[j2], (LANES,)) > 0)
                idx = lax.broadcast(dstl[j2] * H, (LANES,)) + lanes
                plsc.addupdate_scatter(s_loc, [idx], ex, mask=m)
        return 0

    lax.fori_loop(0, EPS // C, chunk, 0)
    pltpu.sync_copy(s_loc, sparts.at[gid])
    plsc.subcore_barrier()

    def reduce_stripe(words, outrows):
        basew = sid * 2560
        _zero1d(redacc, 2560)
        for k in range(NS):
            pltpu.sync_copy(
                sparts.at[cid * NS + k].at[pl.ds(basew, words)],
                redbuf.at[pl.ds(0, words)],
            )

            def add(i, _):
                redacc[pl.ds(i * 16, 16)] = (
                    redacc[pl.ds(i * 16, 16)] + redbuf[pl.ds(i * 16, 16)]
                )
                return 0

            lax.fori_loop(0, words // 16, add, 0)
        pltpu.sync_copy(
            edA.at[pl.ds(half + sid * 320, outrows), :],
            sbuf.at[pl.ds(0, outrows), :],
        )

        def pad(i, _):
            v = redacc[pl.ds(i * 16, 16)]
            gv = 1.0 / (8.0 * (v + 1e-16))
            flat = lax.broadcast(i * 16, (LANES,)) + lanes
            plsc.store_scatter(sbuf, [flat // H, 8 + flat % H], gv)
            return 0

        lax.fori_loop(0, words // 16, pad, 0)
        pltpu.sync_copy(
            sbuf.at[pl.ds(0, outrows), :],
            s_hbm.at[pl.ds(half + sid * 320, outrows), :],
        )

    @pl.when(sid < NS - 1)
    def _():
        reduce_stripe(2560, 320)

    @pl.when(sid == NS - 1)
    def _():
        reduce_stripe(1664, 200)


def _build_sc_s():
    return pl.kernel(
        _sc_s_body,
        out_type=(
            jax.ShapeDtypeStruct((N, 128), f32),
            jax.ShapeDtypeStruct((NC * NS, SLOC), f32),
        ),
        mesh=_mesh,
        compiler_params=_sc_params,
        scratch_types=[
            pltpu.VMEM((C,), i32),
            pltpu.VMEM((C,), i32),
            pltpu.VMEM((C,), i32),
            pltpu.VMEM((C, 128), f32),
            pltpu.VMEM((C, 128), f32),
            pltpu.VMEM((SLOC,), f32),
            pltpu.VMEM((320, 128), f32),
            pltpu.VMEM((2560,), f32),
            pltpu.VMEM((2560,), f32),
            pltpu.SemaphoreType.DMA,
            pltpu.SemaphoreType.DMA,
        ],
        name="gat_sc_s",
    )


def _build_sc_agg(D):
    ROW = H * D
    if D == 128:
        NB, BUCKET = 8, 640
    else:
        NB, BUCKET = 2, 2560
    LAST = HALF - (NB - 1) * BUCKET
    SR = BUCKET // NS
    QD = max(D // LANES, 1)

    def body(srcs, dsts, esA, edA, z, out_hbm, parts,
             src_v, dst_v, pend_s, srck, dstk, dstlk,
             esk, edk, zrows, acc, redbuf, sem1, sem2):
        cid = lax.axis_index("c")
        sid = lax.axis_index("s")
        gid = cid * NS + sid
        half = cid * HALF
        lanes = lax.iota(i32, LANES)
        lm8 = jnp.where(lanes < H, 1.0, 0.0).astype(f32)
        zi16 = jnp.zeros((LANES,), i32)

        for i in range(CAP // 16):
            pend_s[pl.ds(i * 16, 16)] = zi16
        dstlk[pl.ds(0, 16)] = zi16
        dstlk[pl.ds(16, 16)] = zi16

        def batch_fire(base_b, valid_cnt):
            pk = pend_s[pl.ds(0, 16)]
            srck[pl.ds(0, 16)] = pk // 16384
            dstk[pl.ds(0, 16)] = pk % 16384
            cpz = pltpu.async_copy(z.at[srck], zrows, sem1)
            cpe = pltpu.async_copy(esA.at[srck], esk, sem2)
            cpd = pltpu.async_copy(edA.at[dstk], edk, sem2)
            cpz.wait()
            cpe.wait()
            cpd.wait()
            vcnt = lax.broadcast(valid_cnt, (LANES,))
            valid = lanes < vcnt
            dstg = dstk[pl.ds(0, 16)]
            dstlk[pl.ds(0, 16)] = jnp.where(valid, dstg - base_b, 0)

            def mul_j(j, _):
                vd = edk[j, pl.ds(0, 16)]
                e = esk[j, pl.ds(0, 16)] + vd
                e = jnp.where(e > 0, e, 0.2 * e)
                fv = jnp.where(j < valid_cnt, 1.0, 0.0).astype(f32)
                ex = jnp.exp(e) * lm8 * lax.broadcast(fv, (LANES,))
                dj = dstlk[pl.ds(j, 16)][0]
                if D == 128:
                    for q in range(QD):
                        vq = jnp.zeros((LANES,), f32)
                        for h in range(H):
                            av = lax.broadcast(ex[h] * vd[8 + h], (LANES,))
                            vq = vq + av * zrows[j, pl.ds(h * D + q * 16, 16)]
                        sl = pl.ds(q * 16, 16)
                        acc[dj, sl] = acc[dj, sl] + vq
                else:
                    vq = jnp.zeros((LANES,), f32)
                    for h in range(H):
                        av = lax.broadcast(ex[h] * vd[8 + h], (LANES,))
                        vq = vq + av * zrows[j, pl.ds(h * D, 16)]
                    sl = pl.ds(dj * 16, 16)
                    acc[sl] = acc[sl] + vq
                return 0

            lax.fori_loop(0, K, mul_j, 0)

        def bucket(b, _):
            base_b = half + b * BUCKET
            if D == 128:
                _zero2d(acc, BUCKET, D)
            else:
                _zero1d(acc, BUCKET * D)

            def chunk_d(t, cnt):
                off = sid * EPS + t * C
                pltpu.sync_copy(srcs.at[pl.ds(off, C)], src_v)
                pltpu.sync_copy(dsts.at[pl.ds(off, C)], dst_v)
                for g in range(C // 16):
                    srcg = src_v[pl.ds(g * 16, 16)]
                    dstg = dst_v[pl.ds(g * 16, 16)]
                    hit = (
                        (dstg >= base_b)
                        & (dstg < base_b + BUCKET)
                        & (dstg < half + HALF)
                    )
                    pc = plsc.all_reduce_population_count(hit)

                    @pl.when(pc[0] > 0)
                    def _():
                        keys = jnp.where(hit, lanes, lanes + 16)
                        packed = srcg * 16384 + dstg
                        _, pkc = plsc.sort_key_val(keys, packed)
                        pend_s[pl.ds(cnt, 16)] = pkc

                    cnt = cnt + pc[0]

                def fire_body(c):
                    batch_fire(base_b, jnp.int32(K))
                    for i in range(C // 16):
                        pend_s[pl.ds(i * 16, 16)] = pend_s[pl.ds(K + i * 16, 16)]
                    return c - K

                return lax.while_loop(lambda c: c >= K, fire_body, cnt)

            cnt = lax.fori_loop(0, EPS // C, chunk_d, jnp.int32(0))

            @pl.when(cnt > 0)
            def _():
                batch_fire(base_b, cnt)

            pltpu.sync_copy(acc, parts.at[gid])
            plsc.subcore_barrier()

            base_r = sid * SR
            lastb = jnp.where(b == NB - 1, LAST, BUCKET)
            if D == 128:
                _zero2d(acc, SR, D)
                nd16 = D // 16
                for k in range(NS):
                    pltpu.sync_copy(
                        parts.at[cid * NS + k].at[pl.ds(base_r, SR), :],
                        redbuf,
                    )

                    def add(i, _):
                        r = i // nd16
                        c = (i % nd16) * 16
                        acc[r, pl.ds(c, 16)] = (
                            acc[r, pl.ds(c, 16)] + redbuf[r, pl.ds(c, 16)]
                        )
                        return 0

                    lax.fori_loop(0, SR * D // 16, add, 0)

                def dump(i, _):
                    @pl.when(base_r + i * 8 < lastb)
                    def _():
                        pltpu.sync_copy(
                            acc.at[pl.ds(i * 8, 8), :],
                            out_hbm.at[pl.ds(base_b + base_r + i * 8, 8), :],
                        )
                    return 0

                lax.fori_loop(0, SR // 8, dump, 0)
            else:
                _zero1d(acc, SR * 16)
                for k in range(NS):
                    pltpu.sync_copy(
                        parts.at[cid * NS + k].at[pl.ds(base_r * 16, SR * 16)],
                        redbuf,
                    )

                    def add(i, _):
                        acc[pl.ds(i * 16, 16)] = (
                            acc[pl.ds(i * 16, 16)] + redbuf[pl.ds(i * 16, 16)]
                        )
                        return 0

                    lax.fori_loop(0, SR, add, 0)

                def dump(i, _):
                    @pl.when(base_r + i * 8 < lastb)
                    def _():
                        pltpu.sync_copy(
                            acc.at[pl.ds(i * 128, 128)],
                            out_hbm.at[
                                pl.ds((base_b + base_r + i * 8) * 16, 128)
                            ],
                        )
                    return 0

                lax.fori_loop(0, SR // 8, dump, 0)
            plsc.subcore_barrier()
            return 0

        lax.fori_loop(0, NB, bucket, 0)

    if D == 128:
        out_sds = jax.ShapeDtypeStruct((N, D), f32)
        parts_sds = jax.ShapeDtypeStruct((NC * NS, BUCKET, D), f32)
        acc_t = pltpu.VMEM((BUCKET, D), f32)
        red_t = pltpu.VMEM((SR, D), f32)
    else:
        out_sds = jax.ShapeDtypeStruct((N * 16,), f32)
        parts_sds = jax.ShapeDtypeStruct((NC * NS, BUCKET * 16), f32)
        acc_t = pltpu.VMEM((BUCKET * 16,), f32)
        red_t = pltpu.VMEM((SR * 16,), f32)
    return pl.kernel(
        body,
        out_type=(out_sds, parts_sds),
        mesh=_mesh,
        compiler_params=_sc_params,
        scratch_types=[
            pltpu.VMEM((C,), i32),
            pltpu.VMEM((C,), i32),
            pltpu.VMEM((CAP,), i32),
            pltpu.VMEM((K,), i32),
            pltpu.VMEM((K,), i32),
            pltpu.VMEM((2 * K,), i32),
            pltpu.VMEM((K, 128), f32),
            pltpu.VMEM((K, 128), f32),
            pltpu.VMEM((K, ROW), f32),
            acc_t,
            red_t,
            pltpu.SemaphoreType.DMA,
            pltpu.SemaphoreType.DMA,
        ],
        name=f"gat_sc_agg_d{D}",
    )


_sc_s = _build_sc_s()
_sc_agg1 = _build_sc_agg(HID)
_sc_agg2 = _build_sc_agg(NCLS)

BN = 400
GRID = N // BN


def _tc1_body(x_ref, w_ref, a_ref, z_ref, es_ref, ed_ref):
    zv = jnp.dot(x_ref[...], w_ref[...], preferred_element_type=f32)
    z_ref[...] = zv
    zh = zv.reshape(BN, H, HID)
    a = a_ref[...]
    zpad = jnp.zeros((BN, 128 - H), f32)
    es = jnp.sum(zh * a[None, :H, :], axis=-1)
    ed = jnp.sum(zh * a[None, H:, :], axis=-1)
    es_ref[...] = jnp.concatenate([es, zpad], axis=1)
    ed_ref[...] = jnp.concatenate([ed, zpad], axis=1)


def _tc2_body(h_ref, w_ref, a_ref, z_ref, es_ref, ed_ref):
    hmean = h_ref[...]
    hact = jnp.where(hmean > 0, hmean, jnp.exp(hmean) - 1.0)
    zv = jnp.dot(hact, w_ref[...], preferred_element_type=f32)
    z_ref[...] = zv
    zh = zv.reshape(BN, H, NCLS)
    a = a_ref[...]
    zpad = jnp.zeros((BN, 128 - H), f32)
    es = jnp.sum(zh * a[None, :H, :], axis=-1)
    ed = jnp.sum(zh * a[None, H:, :], axis=-1)
    es_ref[...] = jnp.concatenate([es, zpad], axis=1)
    ed_ref[...] = jnp.concatenate([ed, zpad], axis=1)


def _tc3_body(o_ref, out_ref):
    hmean = o_ref[...]
    m = jnp.max(hmean, axis=-1, keepdims=True)
    ev = jnp.exp(hmean - m)
    out_ref[...] = ev / jnp.sum(ev, axis=-1, keepdims=True)


def _row_spec(cols):
    return pl.BlockSpec((BN, cols), lambda i: (i, 0))


def _full_spec(r, c):
    return pl.BlockSpec((r, c), lambda i: (0, 0))


_tc1 = pl.pallas_call(
    _tc1_body,
    grid=(GRID,),
    in_specs=[_row_spec(D_IN), _full_spec(D_IN, H * HID), _full_spec(2 * H, HID)],
    out_specs=[_row_spec(H * HID), _row_spec(128), _row_spec(128)],
    out_shape=[
        jax.ShapeDtypeStruct((N, H * HID), f32),
        jax.ShapeDtypeStruct((N, 128), f32),
        jax.ShapeDtypeStruct((N, 128), f32),
    ],
)

_tc2 = pl.pallas_call(
    _tc2_body,
    grid=(GRID,),
    in_specs=[
        _row_spec(HID),
        _full_spec(HID, H * NCLS),
        _full_spec(2 * H, NCLS),
    ],
    out_specs=[_row_spec(H * NCLS), _row_spec(128), _row_spec(128)],
    out_shape=[
        jax.ShapeDtypeStruct((N, H * NCLS), f32),
        jax.ShapeDtypeStruct((N, 128), f32),
        jax.ShapeDtypeStruct((N, 128), f32),
    ],
)

_tc3 = pl.pallas_call(
    _tc3_body,
    grid=(GRID,),
    in_specs=[_row_spec(NCLS)],
    out_specs=_row_spec(NCLS),
    out_shape=jax.ShapeDtypeStruct((N, NCLS), f32),
)


@jax.jit
def kernel(x, edge_index, W1, a_src1, a_dst1, W2, a_src2, a_dst2):
    srcs = jnp.concatenate(
        [edge_index[0].astype(i32), jnp.zeros((E2 - E,), i32)]
    )
    dsts = jnp.concatenate(
        [edge_index[1].astype(i32), jnp.full((E2 - E,), N, i32)]
    )
    a1 = jnp.concatenate([a_src1, a_dst1], axis=0)
    a2 = jnp.concatenate([a_src2, a_dst2], axis=0)
    z1, es1, ed1 = _tc1(x, W1, a1)
    edg1, _ = _sc_s(srcs, dsts, es1, ed1)
    h1, _ = _sc_agg1(srcs, dsts, es1, edg1, z1)
    z2, es2, ed2 = _tc2(h1, W2, a2)
    edg2, _ = _sc_s(srcs, dsts, es2, ed2)
    o2f, _ = _sc_agg2(srcs, dsts, es2, edg2, z2)
    return _tc3(o2f.reshape(N, NCLS))


# speedup vs baseline: 10.5946x; 1.0899x over previous
import jax
import jax.numpy as jnp
from jax import lax
from jax.experimental import pallas as pl
from jax.experimental.pallas import tpu as pltpu
from jax.experimental.pallas import tpu_sc as plsc

N = 10000
E = 320000
D_IN = 128
HID = 128
NCLS = 16
H = 8

NC = 2
NS = 16
LANES = 16
HALF = N // NC
C = 128
E2 = 327680
EPS = E2 // NS
K = 16
CAP = K + C
SLOC = 40064

f32 = jnp.float32
i32 = jnp.int32

_sc_params = pltpu.CompilerParams(needs_layout_passes=False)
_mesh = plsc.VectorSubcoreMesh(
    core_axis_name="c", subcore_axis_name="s", num_cores=NC, num_subcores=NS
)


def _zero1d(ref, n):
    zf = jnp.zeros((LANES,), ref.dtype)

    def f(i, _):
        ref[pl.ds(i * 16, 16)] = zf
        return 0

    lax.fori_loop(0, n // 16, f, 0)


def _zero2d(ref, rows, cols):
    n16 = cols // 16
    zf = jnp.zeros((LANES,), ref.dtype)

    def f(i, _):
        ref[i // n16, pl.ds((i % n16) * 16, 16)] = zf
        return 0

    lax.fori_loop(0, rows * n16, f, 0)


def _sc_s_body(srcs, dsts, esA, edA, s_hbm, sparts,
               src_v, dst_v, dstc_v, esr, edr, s_loc, sbuf, redacc, redbuf,
               sem1, sem2):
    cid = lax.axis_index("c")
    sid = lax.axis_index("s")
    gid = cid * NS + sid
    half = cid * HALF
    lanes = lax.iota(i32, LANES)
    lt8 = lanes < H
    lm8 = jnp.where(lt8, 1.0, 0.0).astype(f32)

    _zero1d(s_loc, SLOC)

    def chunk(t, _):
        off = sid * EPS + t * C
        pltpu.sync_copy(srcs.at[pl.ds(off, C)], src_v)
        pltpu.sync_copy(dsts.at[pl.ds(off, C)], dst_v)
        for g in range(C // 16):
            sl = pl.ds(g * 16, 16)
            dstc_v[sl] = jnp.minimum(dst_v[sl], N - 1)
        cp1 = pltpu.async_copy(esA.at[src_v], esr, sem1)
        cp2 = pltpu.async_copy(edA.at[dstc_v], edr, sem2)
        cp1.wait()
        cp2.wait()
        for g in range(C // 16):
            dstg = dst_v[pl.ds(g * 16, 16)]
            hit = (dstg >= half) & (dstg < half + HALF)
            hiti = jnp.where(hit, 1, 0).astype(i32)
            dstl = jnp.where(hit, dstg - half, 0)
            for j2 in range(16):
                j = g * 16 + j2
                e = esr[j, pl.ds(0, 16)] + edr[j, pl.ds(0, 16)]
                e = jnp.where(e > 0, e, 0.2 * e)
                ex = jnp.exp(e) * lm8
                m = lt8 & (lax.broadcast(hiti[j2], (LANES,)) > 0)
                idx = lax.broadcast(dstl[j2] * H, (LANES,)) + lanes
                plsc.addupdate_scatter(s_loc, [idx], ex, mask=m)
        return 0

    lax.fori_loop(0, EPS // C, chunk, 0)
    pltpu.sync_copy(s_loc, sparts.at[gid])
    plsc.subcore_barrier()

    def reduce_stripe(words, outrows):
        basew = sid * 2560
        _zero1d(redacc, 2560)
        for k in range(NS):
            pltpu.sync_copy(
                sparts.at[cid * NS + k].at[pl.ds(basew, words)],
                redbuf.at[pl.ds(0, words)],
            )

            def add(i, _):
                redacc[pl.ds(i * 16, 16)] = (
                    redacc[pl.ds(i * 16, 16)] + redbuf[pl.ds(i * 16, 16)]
                )
                return 0

            lax.fori_loop(0, words // 16, add, 0)
        pltpu.sync_copy(
            edA.at[pl.ds(half + sid * 320, outrows), :],
            sbuf.at[pl.ds(0, outrows), :],
        )

        def pad(i, _):
            v = redacc[pl.ds(i * 16, 16)]
            gv = 1.0 / (8.0 * (v + 1e-16))
            flat = lax.broadcast(i * 16, (LANES,)) + lanes
            plsc.store_scatter(sbuf, [flat // H, 8 + flat % H], gv)
            return 0

        lax.fori_loop(0, words // 16, pad, 0)
        pltpu.sync_copy(
            sbuf.at[pl.ds(0, outrows), :],
            s_hbm.at[pl.ds(half + sid * 320, outrows), :],
        )

    @pl.when(sid < NS - 1)
    def _():
        reduce_stripe(2560, 320)

    @pl.when(sid == NS - 1)
    def _():
        reduce_stripe(1664, 200)


def _build_sc_s():
    return pl.kernel(
        _sc_s_body,
        out_type=(
            jax.ShapeDtypeStruct((N, 128), f32),
            jax.ShapeDtypeStruct((NC * NS, SLOC), f32),
        ),
        mesh=_mesh,
        compiler_params=_sc_params,
        scratch_types=[
            pltpu.VMEM((C,), i32),
            pltpu.VMEM((C,), i32),
            pltpu.VMEM((C,), i32),
            pltpu.VMEM((C, 128), f32),
            pltpu.VMEM((C, 128), f32),
            pltpu.VMEM((SLOC,), f32),
            pltpu.VMEM((320, 128), f32),
            pltpu.VMEM((2560,), f32),
            pltpu.VMEM((2560,), f32),
            pltpu.SemaphoreType.DMA,
            pltpu.SemaphoreType.DMA,
        ],
        name="gat_sc_s",
    )


def _build_sc_agg(D):
    ROW = H * D
    if D == 128:
        NB, BUCKET = 8, 640
    else:
        NB, BUCKET = 2, 2560
    LAST = HALF - (NB - 1) * BUCKET
    SR = BUCKET // NS
    QD = max(D // LANES, 1)
    CA = 256
    CAPA = K + CA

    def body(srcs, dsts, esA, edA, z, out_hbm, parts,
             src_v, dst_v, pend_s, srck, dstk, dstlk,
             esk, edk, zrows, acc, redbuf, sem1, sem2):
        cid = lax.axis_index("c")
        sid = lax.axis_index("s")
        gid = cid * NS + sid
        half = cid * HALF
        lanes = lax.iota(i32, LANES)
        lm8 = jnp.where(lanes < H, 1.0, 0.0).astype(f32)
        zi16 = jnp.zeros((LANES,), i32)

        for i in range(CAPA // 16):
            pend_s[pl.ds(i * 16, 16)] = zi16
        dstlk[pl.ds(0, 16)] = zi16
        dstlk[pl.ds(16, 16)] = zi16

        def batch_fire(base_b, valid_cnt):
            pk = pend_s[pl.ds(0, 16)]
            srck[pl.ds(0, 16)] = pk // 16384
            dstk[pl.ds(0, 16)] = pk % 16384
            cpz = pltpu.async_copy(z.at[srck], zrows, sem1)
            cpe = pltpu.async_copy(esA.at[srck], esk, sem2)
            cpd = pltpu.async_copy(edA.at[dstk], edk, sem2)
            cpz.wait()
            cpe.wait()
            cpd.wait()
            vcnt = lax.broadcast(valid_cnt, (LANES,))
            valid = lanes < vcnt
            dstg = dstk[pl.ds(0, 16)]
            dstlk[pl.ds(0, 16)] = jnp.where(valid, dstg - base_b, 0)

            def mul_j(j, _):
                vd = edk[j, pl.ds(0, 16)]
                e = esk[j, pl.ds(0, 16)] + vd
                e = jnp.where(e > 0, e, 0.2 * e)
                fv = jnp.where(j < valid_cnt, 1.0, 0.0).astype(f32)
                ex = jnp.exp(e) * lm8 * lax.broadcast(fv, (LANES,))
                dj = dstlk[pl.ds(j, 16)][0]
                if D == 128:
                    for q in range(QD):
                        vq = jnp.zeros((LANES,), f32)
                        for h in range(H):
                            av = lax.broadcast(ex[h] * vd[8 + h], (LANES,))
                            vq = vq + av * zrows[j, pl.ds(h * D + q * 16, 16)]
                        sl = pl.ds(q * 16, 16)
                        acc[dj, sl] = acc[dj, sl] + vq
                else:
                    vq = jnp.zeros((LANES,), f32)
                    for h in range(H):
                        av = lax.broadcast(ex[h] * vd[8 + h], (LANES,))
                        vq = vq + av * zrows[j, pl.ds(h * D, 16)]
                    sl = pl.ds(dj * 16, 16)
                    acc[sl] = acc[sl] + vq
                return 0

            lax.fori_loop(0, K, mul_j, 0)

        def bucket(b, _):
            base_b = half + b * BUCKET
            if D == 128:
                _zero2d(acc, BUCKET, D)
            else:
                _zero1d(acc, BUCKET * D)

            def chunk_d(t, cnt):
                off = sid * EPS + t * CA
                pltpu.sync_copy(srcs.at[pl.ds(off, CA)], src_v)
                pltpu.sync_copy(dsts.at[pl.ds(off, CA)], dst_v)
                for g in range(CA // 16):
                    srcg = src_v[pl.ds(g * 16, 16)]
                    dstg = dst_v[pl.ds(g * 16, 16)]
                    hit = (
                        (dstg >= base_b)
                        & (dstg < base_b + BUCKET)
                        & (dstg < half + HALF)
                    )
                    pc = plsc.all_reduce_population_count(hit)

                    @pl.when(pc[0] > 0)
                    def _():
                        keys = jnp.where(hit, lanes, lanes + 16)
                        packed = srcg * 16384 + dstg
                        _, pkc = plsc.sort_key_val(keys, packed)
                        pend_s[pl.ds(cnt, 16)] = pkc

                    cnt = cnt + pc[0]

                def fire_body(c):
                    batch_fire(base_b, jnp.int32(K))
                    for i in range(CA // 16):
                        pend_s[pl.ds(i * 16, 16)] = pend_s[pl.ds(K + i * 16, 16)]
                    return c - K

                return lax.while_loop(lambda c: c >= K, fire_body, cnt)

            cnt = lax.fori_loop(0, EPS // CA, chunk_d, jnp.int32(0))

            @pl.when(cnt > 0)
            def _():
                batch_fire(base_b, cnt)

            pltpu.sync_copy(acc, parts.at[gid])
            plsc.subcore_barrier()

            base_r = sid * SR
            lastb = jnp.where(b == NB - 1, LAST, BUCKET)
            if D == 128:
                _zero2d(acc, SR, D)
                nd16 = D // 16
                for k in range(NS):
                    pltpu.sync_copy(
                        parts.at[cid * NS + k].at[pl.ds(base_r, SR), :],
                        redbuf,
                    )

                    def add(i, _):
                        r = i // nd16
                        c = (i % nd16) * 16
                        acc[r, pl.ds(c, 16)] = (
                            acc[r, pl.ds(c, 16)] + redbuf[r, pl.ds(c, 16)]
                        )
                        return 0

                    lax.fori_loop(0, SR * D // 16, add, 0)

                def dump(i, _):
                    @pl.when(base_r + i * 8 < lastb)
                    def _():
                        pltpu.sync_copy(
                            acc.at[pl.ds(i * 8, 8), :],
                            out_hbm.at[pl.ds(base_b + base_r + i * 8, 8), :],
                        )
                    return 0

                lax.fori_loop(0, SR // 8, dump, 0)
            else:
                _zero1d(acc, SR * 16)
                for k in range(NS):
                    pltpu.sync_copy(
                        parts.at[cid * NS + k].at[pl.ds(base_r * 16, SR * 16)],
                        redbuf,
                    )

                    def add(i, _):
                        acc[pl.ds(i * 16, 16)] = (
                            acc[pl.ds(i * 16, 16)] + redbuf[pl.ds(i * 16, 16)]
                        )
                        return 0

                    lax.fori_loop(0, SR, add, 0)

                def dump(i, _):
                    @pl.when(base_r + i * 8 < lastb)
                    def _():
                        pltpu.sync_copy(
                            acc.at[pl.ds(i * 128, 128)],
                            out_hbm.at[
                                pl.ds((base_b + base_r + i * 8) * 16, 128)
                            ],
                        )
                    return 0

                lax.fori_loop(0, SR // 8, dump, 0)
            plsc.subcore_barrier()
            return 0

        lax.fori_loop(0, NB, bucket, 0)

    if D == 128:
        out_sds = jax.ShapeDtypeStruct((N, D), f32)
        parts_sds = jax.ShapeDtypeStruct((NC * NS, BUCKET, D), f32)
        acc_t = pltpu.VMEM((BUCKET, D), f32)
        red_t = pltpu.VMEM((SR, D), f32)
    else:
        out_sds = jax.ShapeDtypeStruct((N * 16,), f32)
        parts_sds = jax.ShapeDtypeStruct((NC * NS, BUCKET * 16), f32)
        acc_t = pltpu.VMEM((BUCKET * 16,), f32)
        red_t = pltpu.VMEM((SR * 16,), f32)
    return pl.kernel(
        body,
        out_type=(out_sds, parts_sds),
        mesh=_mesh,
        compiler_params=_sc_params,
        scratch_types=[
            pltpu.VMEM((CA,), i32),
            pltpu.VMEM((CA,), i32),
            pltpu.VMEM((CAPA,), i32),
            pltpu.VMEM((K,), i32),
            pltpu.VMEM((K,), i32),
            pltpu.VMEM((2 * K,), i32),
            pltpu.VMEM((K, 128), f32),
            pltpu.VMEM((K, 128), f32),
            pltpu.VMEM((K, ROW), f32),
            acc_t,
            red_t,
            pltpu.SemaphoreType.DMA,
            pltpu.SemaphoreType.DMA,
        ],
        name=f"gat_sc_agg_d{D}",
    )


_sc_s = _build_sc_s()
_sc_agg1 = _build_sc_agg(HID)
_sc_agg2 = _build_sc_agg(NCLS)

BN = 400
GRID = N // BN


def _tc1_body(x_ref, w_ref, a_ref, z_ref, es_ref, ed_ref):
    zv = jnp.dot(x_ref[...], w_ref[...], preferred_element_type=f32)
    z_ref[...] = zv
    zh = zv.reshape(BN, H, HID)
    a = a_ref[...]
    zpad = jnp.zeros((BN, 128 - H), f32)
    es = jnp.sum(zh * a[None, :H, :], axis=-1)
    ed = jnp.sum(zh * a[None, H:, :], axis=-1)
    es_ref[...] = jnp.concatenate([es, zpad], axis=1)
    ed_ref[...] = jnp.concatenate([ed, zpad], axis=1)


def _tc2_body(h_ref, w_ref, a_ref, z_ref, es_ref, ed_ref):
    hmean = h_ref[...]
    hact = jnp.where(hmean > 0, hmean, jnp.exp(hmean) - 1.0)
    zv = jnp.dot(hact, w_ref[...], preferred_element_type=f32)
    z_ref[...] = zv
    zh = zv.reshape(BN, H, NCLS)
    a = a_ref[...]
    zpad = jnp.zeros((BN, 128 - H), f32)
    es = jnp.sum(zh * a[None, :H, :], axis=-1)
    ed = jnp.sum(zh * a[None, H:, :], axis=-1)
    es_ref[...] = jnp.concatenate([es, zpad], axis=1)
    ed_ref[...] = jnp.concatenate([ed, zpad], axis=1)


def _tc3_body(o_ref, out_ref):
    hmean = o_ref[...]
    m = jnp.max(hmean, axis=-1, keepdims=True)
    ev = jnp.exp(hmean - m)
    out_ref[...] = ev / jnp.sum(ev, axis=-1, keepdims=True)


def _row_spec(cols):
    return pl.BlockSpec((BN, cols), lambda i: (i, 0))


def _full_spec(r, c):
    return pl.BlockSpec((r, c), lambda i: (0, 0))


_tc1 = pl.pallas_call(
    _tc1_body,
    grid=(GRID,),
    in_specs=[_row_spec(D_IN), _full_spec(D_IN, H * HID), _full_spec(2 * H, HID)],
    out_specs=[_row_spec(H * HID), _row_spec(128), _row_spec(128)],
    out_shape=[
        jax.ShapeDtypeStruct((N, H * HID), f32),
        jax.ShapeDtypeStruct((N, 128), f32),
        jax.ShapeDtypeStruct((N, 128), f32),
    ],
)

_tc2 = pl.pallas_call(
    _tc2_body,
    grid=(GRID,),
    in_specs=[
        _row_spec(HID),
        _full_spec(HID, H * NCLS),
        _full_spec(2 * H, NCLS),
    ],
    out_specs=[_row_spec(H * NCLS), _row_spec(128), _row_spec(128)],
    out_shape=[
        jax.ShapeDtypeStruct((N, H * NCLS), f32),
        jax.ShapeDtypeStruct((N, 128), f32),
        jax.ShapeDtypeStruct((N, 128), f32),
    ],
)

_tc3 = pl.pallas_call(
    _tc3_body,
    grid=(GRID,),
    in_specs=[_row_spec(NCLS)],
    out_specs=_row_spec(NCLS),
    out_shape=jax.ShapeDtypeStruct((N, NCLS), f32),
)


@jax.jit
def kernel(x, edge_index, W1, a_src1, a_dst1, W2, a_src2, a_dst2):
    srcs = jnp.concatenate(
        [edge_index[0].astype(i32), jnp.zeros((E2 - E,), i32)]
    )
    dsts = jnp.concatenate(
        [edge_index[1].astype(i32), jnp.full((E2 - E,), N, i32)]
    )
    a1 = jnp.concatenate([a_src1, a_dst1], axis=0)
    a2 = jnp.concatenate([a_src2, a_dst2], axis=0)
    z1, es1, ed1 = _tc1(x, W1, a1)
    edg1, _ = _sc_s(srcs, dsts, es1, ed1)
    h1, _ = _sc_agg1(srcs, dsts, es1, edg1, z1)
    z2, es2, ed2 = _tc2(h1, W2, a2)
    edg2, _ = _sc_s(srcs, dsts, es2, ed2)
    o2f, _ = _sc_agg2(srcs, dsts, es2, edg2, z2)
    return _tc3(o2f.reshape(N, NCLS))


# speedup vs baseline: 11.0232x; 1.0404x over previous
import jax
import jax.numpy as jnp
from jax import lax
from jax.experimental import pallas as pl
from jax.experimental.pallas import tpu as pltpu
from jax.experimental.pallas import tpu_sc as plsc

N = 10000
E = 320000
D_IN = 128
HID = 128
NCLS = 16
H = 8

NC = 2
NS = 16
LANES = 16
HALF = N // NC
C = 128
E2 = 327680
EPS = E2 // NS
K = 16
CAP = K + C
SLOC = 40064

f32 = jnp.float32
i32 = jnp.int32

_sc_params = pltpu.CompilerParams(needs_layout_passes=False)
_mesh = plsc.VectorSubcoreMesh(
    core_axis_name="c", subcore_axis_name="s", num_cores=NC, num_subcores=NS
)


def _zero1d(ref, n):
    zf = jnp.zeros((LANES,), ref.dtype)

    def f(i, _):
        ref[pl.ds(i * 16, 16)] = zf
        return 0

    lax.fori_loop(0, n // 16, f, 0)


def _zero2d(ref, rows, cols):
    n16 = cols // 16
    zf = jnp.zeros((LANES,), ref.dtype)

    def f(i, _):
        ref[i // n16, pl.ds((i % n16) * 16, 16)] = zf
        return 0

    lax.fori_loop(0, rows * n16, f, 0)


def _sc_s_body(srcs, dsts, esA, edA, s_hbm, sparts,
               src_v, dst_v, dstc_v, esr, edr, s_loc, sbuf, redacc, redbuf,
               sem1, sem2):
    cid = lax.axis_index("c")
    sid = lax.axis_index("s")
    gid = cid * NS + sid
    half = cid * HALF
    lanes = lax.iota(i32, LANES)
    lt8 = lanes < H
    lm8 = jnp.where(lt8, 1.0, 0.0).astype(f32)

    _zero1d(s_loc, SLOC)

    def chunk(t, _):
        off = sid * EPS + t * C
        pltpu.sync_copy(srcs.at[pl.ds(off, C)], src_v)
        pltpu.sync_copy(dsts.at[pl.ds(off, C)], dst_v)
        for g in range(C // 16):
            sl = pl.ds(g * 16, 16)
            dstc_v[sl] = jnp.minimum(dst_v[sl], N - 1)
        cp1 = pltpu.async_copy(esA.at[src_v], esr, sem1)
        cp2 = pltpu.async_copy(edA.at[dstc_v], edr, sem2)
        cp1.wait()
        cp2.wait()
        for g in range(C // 16):
            dstg = dst_v[pl.ds(g * 16, 16)]
            hit = (dstg >= half) & (dstg < half + HALF)
            hiti = jnp.where(hit, 1, 0).astype(i32)
            dstl = jnp.where(hit, dstg - half, 0)
            for j2 in range(16):
                j = g * 16 + j2
                e = esr[j, pl.ds(0, 16)] + edr[j, pl.ds(0, 16)]
                e = jnp.where(e > 0, e, 0.2 * e)
                ex = jnp.exp(e) * lm8
                m = lt8 & (lax.broadcast(hiti[j2], (LANES,)) > 0)
                idx = lax.broadcast(dstl[j2] * H, (LANES,)) + lanes
                plsc.addupdate_scatter(s_loc, [idx], ex, mask=m)
        return 0

    lax.fori_loop(0, EPS // C, chunk, 0)
    pltpu.sync_copy(s_loc, sparts.at[gid])
    plsc.subcore_barrier()

    def reduce_stripe(words, outrows):
        basew = sid * 2560
        _zero1d(redacc, 2560)
        for k in range(NS):
            pltpu.sync_copy(
                sparts.at[cid * NS + k].at[pl.ds(basew, words)],
                redbuf.at[pl.ds(0, words)],
            )

            def add(i, _):
                redacc[pl.ds(i * 16, 16)] = (
                    redacc[pl.ds(i * 16, 16)] + redbuf[pl.ds(i * 16, 16)]
                )
                return 0

            lax.fori_loop(0, words // 16, add, 0)
        pltpu.sync_copy(
            edA.at[pl.ds(half + sid * 320, outrows), :],
            sbuf.at[pl.ds(0, outrows), :],
        )

        def pad(i, _):
            v = redacc[pl.ds(i * 16, 16)]
            gv = 1.0 / (8.0 * (v + 1e-16))
            flat = lax.broadcast(i * 16, (LANES,)) + lanes
            plsc.store_scatter(sbuf, [flat // H, 8 + flat % H], gv)
            return 0

        lax.fori_loop(0, words // 16, pad, 0)
        pltpu.sync_copy(
            sbuf.at[pl.ds(0, outrows), :],
            s_hbm.at[pl.ds(half + sid * 320, outrows), :],
        )

    @pl.when(sid < NS - 1)
    def _():
        reduce_stripe(2560, 320)

    @pl.when(sid == NS - 1)
    def _():
        reduce_stripe(1664, 200)


def _build_sc_s():
    return pl.kernel(
        _sc_s_body,
        out_type=(
            jax.ShapeDtypeStruct((N, 128), f32),
            jax.ShapeDtypeStruct((NC * NS, SLOC), f32),
        ),
        mesh=_mesh,
        compiler_params=_sc_params,
        scratch_types=[
            pltpu.VMEM((C,), i32),
            pltpu.VMEM((C,), i32),
            pltpu.VMEM((C,), i32),
            pltpu.VMEM((C, 128), f32),
            pltpu.VMEM((C, 128), f32),
            pltpu.VMEM((SLOC,), f32),
            pltpu.VMEM((320, 128), f32),
            pltpu.VMEM((2560,), f32),
            pltpu.VMEM((2560,), f32),
            pltpu.SemaphoreType.DMA,
            pltpu.SemaphoreType.DMA,
        ],
        name="gat_sc_s",
    )


def _build_sc_agg(D):
    ROW = H * D
    if D == 128:
        NB, BUCKET = 8, 640
    else:
        NB, BUCKET = 2, 2560
    LAST = HALF - (NB - 1) * BUCKET
    SR = BUCKET // NS
    QD = max(D // LANES, 1)
    CA = 256
    KD = 32
    CAPA = KD + CA

    def body(srcs, dsts, esA, edA, z, out_hbm, parts,
             src_v, dst_v, pend_s, srck, dstk, dstlk,
             esk, edk, zrows, acc, redbuf, sem1, sem2):
        cid = lax.axis_index("c")
        sid = lax.axis_index("s")
        gid = cid * NS + sid
        half = cid * HALF
        lanes = lax.iota(i32, LANES)
        lm8 = jnp.where(lanes < H, 1.0, 0.0).astype(f32)
        zi16 = jnp.zeros((LANES,), i32)

        for i in range(CAPA // 16):
            pend_s[pl.ds(i * 16, 16)] = zi16
        dstlk[pl.ds(0, 16)] = zi16
        dstlk[pl.ds(16, 16)] = zi16

        def batch_fire(base_b, valid_cnt):
            for i in range(KD // 16):
                pk = pend_s[pl.ds(i * 16, 16)]
                srck[pl.ds(i * 16, 16)] = pk // 16384
                dstk[pl.ds(i * 16, 16)] = pk % 16384
            cpz = pltpu.async_copy(z.at[srck], zrows, sem1)
            cpe = pltpu.async_copy(esA.at[srck], esk, sem2)
            cpd = pltpu.async_copy(edA.at[dstk], edk, sem2)
            cpz.wait()
            cpe.wait()
            cpd.wait()
            vcnt = lax.broadcast(valid_cnt, (LANES,))
            for i in range(KD // 16):
                valid = (lanes + i * 16) < vcnt
                dstg = dstk[pl.ds(i * 16, 16)]
                dstlk[pl.ds(i * 16, 16)] = jnp.where(valid, dstg - base_b, 0)

            def mul_j(j, _):
                vd = edk[j, pl.ds(0, 16)]
                e = esk[j, pl.ds(0, 16)] + vd
                e = jnp.where(e > 0, e, 0.2 * e)
                fv = jnp.where(j < valid_cnt, 1.0, 0.0).astype(f32)
                ex = jnp.exp(e) * lm8 * lax.broadcast(fv, (LANES,))
                dj = dstlk[pl.ds(j, 16)][0]
                if D == 128:
                    for q in range(QD):
                        vq = jnp.zeros((LANES,), f32)
                        for h in range(H):
                            av = lax.broadcast(ex[h] * vd[8 + h], (LANES,))
                            vq = vq + av * zrows[j, pl.ds(h * D + q * 16, 16)]
                        sl = pl.ds(q * 16, 16)
                        acc[dj, sl] = acc[dj, sl] + vq
                else:
                    vq = jnp.zeros((LANES,), f32)
                    for h in range(H):
                        av = lax.broadcast(ex[h] * vd[8 + h], (LANES,))
                        vq = vq + av * zrows[j, pl.ds(h * D, 16)]
                    sl = pl.ds(dj * 16, 16)
                    acc[sl] = acc[sl] + vq
                return 0

            lax.fori_loop(0, KD, mul_j, 0)

        def bucket(b, _):
            base_b = half + b * BUCKET
            if D == 128:
                _zero2d(acc, BUCKET, D)
            else:
                _zero1d(acc, BUCKET * D)

            def chunk_d(t, cnt):
                off = sid * EPS + t * CA
                pltpu.sync_copy(srcs.at[pl.ds(off, CA)], src_v)
                pltpu.sync_copy(dsts.at[pl.ds(off, CA)], dst_v)
                for g in range(CA // 16):
                    srcg = src_v[pl.ds(g * 16, 16)]
                    dstg = dst_v[pl.ds(g * 16, 16)]
                    hit = (
                        (dstg >= base_b)
                        & (dstg < base_b + BUCKET)
                        & (dstg < half + HALF)
                    )
                    pc = plsc.all_reduce_population_count(hit)

                    @pl.when(pc[0] > 0)
                    def _():
                        keys = jnp.where(hit, lanes, lanes + 16)
                        packed = srcg * 16384 + dstg
                        _, pkc = plsc.sort_key_val(keys, packed)
                        pend_s[pl.ds(cnt, 16)] = pkc

                    cnt = cnt + pc[0]

                def fire_body(c):
                    batch_fire(base_b, jnp.int32(KD))
                    for i in range(CA // 16):
                        pend_s[pl.ds(i * 16, 16)] = pend_s[pl.ds(KD + i * 16, 16)]
                    return c - KD

                return lax.while_loop(lambda c: c >= KD, fire_body, cnt)

            cnt = lax.fori_loop(0, EPS // CA, chunk_d, jnp.int32(0))

            @pl.when(cnt > 0)
            def _():
                batch_fire(base_b, cnt)

            pltpu.sync_copy(acc, parts.at[gid])
            plsc.subcore_barrier()

            base_r = sid * SR
            lastb = jnp.where(b == NB - 1, LAST, BUCKET)
            if D == 128:
                _zero2d(acc, SR, D)
                nd16 = D // 16
                for k in range(NS):
                    pltpu.sync_copy(
                        parts.at[cid * NS + k].at[pl.ds(base_r, SR), :],
                        redbuf,
                    )

                    def add(i, _):
                        r = i // nd16
                        c = (i % nd16) * 16
                        acc[r, pl.ds(c, 16)] = (
                            acc[r, pl.ds(c, 16)] + redbuf[r, pl.ds(c, 16)]
                        )
                        return 0

                    lax.fori_loop(0, SR * D // 16, add, 0)

                def dump(i, _):
                    @pl.when(base_r + i * 8 < lastb)
                    def _():
                        pltpu.sync_copy(
                            acc.at[pl.ds(i * 8, 8), :],
                            out_hbm.at[pl.ds(base_b + base_r + i * 8, 8), :],
                        )
                    return 0

                lax.fori_loop(0, SR // 8, dump, 0)
            else:
                _zero1d(acc, SR * 16)
                for k in range(NS):
                    pltpu.sync_copy(
                        parts.at[cid * NS + k].at[pl.ds(base_r * 16, SR * 16)],
                        redbuf,
                    )

                    def add(i, _):
                        acc[pl.ds(i * 16, 16)] = (
                            acc[pl.ds(i * 16, 16)] + redbuf[pl.ds(i * 16, 16)]
                        )
                        return 0

                    lax.fori_loop(0, SR, add, 0)

                def dump(i, _):
                    @pl.when(base_r + i * 8 < lastb)
                    def _():
                        pltpu.sync_copy(
                            acc.at[pl.ds(i * 128, 128)],
                            out_hbm.at[
                                pl.ds((base_b + base_r + i * 8) * 16, 128)
                            ],
                        )
                    return 0

                lax.fori_loop(0, SR // 8, dump, 0)
            plsc.subcore_barrier()
            return 0

        lax.fori_loop(0, NB, bucket, 0)

    if D == 128:
        out_sds = jax.ShapeDtypeStruct((N, D), f32)
        parts_sds = jax.ShapeDtypeStruct((NC * NS, BUCKET, D), f32)
        acc_t = pltpu.VMEM((BUCKET, D), f32)
        red_t = pltpu.VMEM((SR, D), f32)
    else:
        out_sds = jax.ShapeDtypeStruct((N * 16,), f32)
        parts_sds = jax.ShapeDtypeStruct((NC * NS, BUCKET * 16), f32)
        acc_t = pltpu.VMEM((BUCKET * 16,), f32)
        red_t = pltpu.VMEM((SR * 16,), f32)
    return pl.kernel(
        body,
        out_type=(out_sds, parts_sds),
        mesh=_mesh,
        compiler_params=_sc_params,
        scratch_types=[
            pltpu.VMEM((CA,), i32),
            pltpu.VMEM((CA,), i32),
            pltpu.VMEM((CAPA,), i32),
            pltpu.VMEM((KD,), i32),
            pltpu.VMEM((KD,), i32),
            pltpu.VMEM((KD + 16,), i32),
            pltpu.VMEM((KD, 128), f32),
            pltpu.VMEM((KD, 128), f32),
            pltpu.VMEM((KD, ROW), f32),
            acc_t,
            red_t,
            pltpu.SemaphoreType.DMA,
            pltpu.SemaphoreType.DMA,
        ],
        name=f"gat_sc_agg_d{D}",
    )


_sc_s = _build_sc_s()
_sc_agg1 = _build_sc_agg(HID)
_sc_agg2 = _build_sc_agg(NCLS)

BN = 400
GRID = N // BN


def _tc1_body(x_ref, w_ref, a_ref, z_ref, es_ref, ed_ref):
    zv = jnp.dot(x_ref[...], w_ref[...], preferred_element_type=f32)
    z_ref[...] = zv
    zh = zv.reshape(BN, H, HID)
    a = a_ref[...]
    zpad = jnp.zeros((BN, 128 - H), f32)
    es = jnp.sum(zh * a[None, :H, :], axis=-1)
    ed = jnp.sum(zh * a[None, H:, :], axis=-1)
    es_ref[...] = jnp.concatenate([es, zpad], axis=1)
    ed_ref[...] = jnp.concatenate([ed, zpad], axis=1)


def _tc2_body(h_ref, w_ref, a_ref, z_ref, es_ref, ed_ref):
    hmean = h_ref[...]
    hact = jnp.where(hmean > 0, hmean, jnp.exp(hmean) - 1.0)
    zv = jnp.dot(hact, w_ref[...], preferred_element_type=f32)
    z_ref[...] = zv
    zh = zv.reshape(BN, H, NCLS)
    a = a_ref[...]
    zpad = jnp.zeros((BN, 128 - H), f32)
    es = jnp.sum(zh * a[None, :H, :], axis=-1)
    ed = jnp.sum(zh * a[None, H:, :], axis=-1)
    es_ref[...] = jnp.concatenate([es, zpad], axis=1)
    ed_ref[...] = jnp.concatenate([ed, zpad], axis=1)


def _tc3_body(o_ref, out_ref):
    hmean = o_ref[...]
    m = jnp.max(hmean, axis=-1, keepdims=True)
    ev = jnp.exp(hmean - m)
    out_ref[...] = ev / jnp.sum(ev, axis=-1, keepdims=True)


def _row_spec(cols):
    return pl.BlockSpec((BN, cols), lambda i: (i, 0))


def _full_spec(r, c):
    return pl.BlockSpec((r, c), lambda i: (0, 0))


_tc1 = pl.pallas_call(
    _tc1_body,
    grid=(GRID,),
    in_specs=[_row_spec(D_IN), _full_spec(D_IN, H * HID), _full_spec(2 * H, HID)],
    out_specs=[_row_spec(H * HID), _row_spec(128), _row_spec(128)],
    out_shape=[
        jax.ShapeDtypeStruct((N, H * HID), f32),
        jax.ShapeDtypeStruct((N, 128), f32),
        jax.ShapeDtypeStruct((N, 128), f32),
    ],
)

_tc2 = pl.pallas_call(
    _tc2_body,
    grid=(GRID,),
    in_specs=[
        _row_spec(HID),
        _full_spec(HID, H * NCLS),
        _full_spec(2 * H, NCLS),
    ],
    out_specs=[_row_spec(H * NCLS), _row_spec(128), _row_spec(128)],
    out_shape=[
        jax.ShapeDtypeStruct((N, H * NCLS), f32),
        jax.ShapeDtypeStruct((N, 128), f32),
        jax.ShapeDtypeStruct((N, 128), f32),
    ],
)

_tc3 = pl.pallas_call(
    _tc3_body,
    grid=(GRID,),
    in_specs=[_row_spec(NCLS)],
    out_specs=_row_spec(NCLS),
    out_shape=jax.ShapeDtypeStruct((N, NCLS), f32),
)


@jax.jit
def kernel(x, edge_index, W1, a_src1, a_dst1, W2, a_src2, a_dst2):
    srcs = jnp.concatenate(
        [edge_index[0].astype(i32), jnp.zeros((E2 - E,), i32)]
    )
    dsts = jnp.concatenate(
        [edge_index[1].astype(i32), jnp.full((E2 - E,), N, i32)]
    )
    a1 = jnp.concatenate([a_src1, a_dst1], axis=0)
    a2 = jnp.concatenate([a_src2, a_dst2], axis=0)
    z1, es1, ed1 = _tc1(x, W1, a1)
    edg1, _ = _sc_s(srcs, dsts, es1, ed1)
    h1, _ = _sc_agg1(srcs, dsts, es1, edg1, z1)
    z2, es2, ed2 = _tc2(h1, W2, a2)
    edg2, _ = _sc_s(srcs, dsts, es2, ed2)
    o2f, _ = _sc_agg2(srcs, dsts, es2, edg2, z2)
    return _tc3(o2f.reshape(N, NCLS))
